# Initial kernel scaffold; baseline (speedup 1.0000x reference)
import jax
import jax.numpy as jnp
from jax.experimental import pallas as pl


def kernel(x, W):
    raise NotImplementedError("write your pallas kernel here")


# speedup vs baseline: 5.0499x; 5.0499x over previous
import jax
import jax.numpy as jnp
from jax.experimental import pallas as pl


def _bcast_body(w_ref, o_ref):
    o_ref[...] = jnp.broadcast_to(w_ref[...][None, :, :], o_ref.shape)


def kernel(x, W):
    B, S = x.shape
    D = W.shape[1]
    SBLK = 512
    assert S % SBLK == 0
    out = pl.pallas_call(
        _bcast_body,
        grid=(S // SBLK,),
        in_specs=[pl.BlockSpec((SBLK, D), lambda s: (s, 0))],
        out_specs=pl.BlockSpec((B, SBLK, D), lambda s: (0, s, 0)),
        out_shape=jax.ShapeDtypeStruct((B, S, D), W.dtype),
    )(W[:S])
    return out


# speedup vs baseline: 5.1809x; 1.0259x over previous
import jax
import jax.numpy as jnp
from jax.experimental import pallas as pl


def _bcast_body(w_ref, o_ref):
    o_ref[...] = jnp.broadcast_to(w_ref[...][None, :, :], o_ref.shape)


def kernel(x, W):
    B, S = x.shape
    D = W.shape[1]
    SBLK = 1024
    assert S % SBLK == 0
    out = pl.pallas_call(
        _bcast_body,
        grid=(S // SBLK,),
        in_specs=[pl.BlockSpec((SBLK, D), lambda s: (s, 0))],
        out_specs=pl.BlockSpec((B, SBLK, D), lambda s: (0, s, 0)),
        out_shape=jax.ShapeDtypeStruct((B, S, D), W.dtype),
    )(W[:S])
    return out
